# baseline (speedup 1.0000x reference)
import functools

import jax
import jax.numpy as jnp
from jax import lax
from jax.experimental import pallas as pl
from jax.experimental.pallas import tpu as pltpu
from jax.experimental.pallas import tpu_sc as plsc

N = 10000
F = 128
FH = F // 2
E = 320000
K = 15
DRUG_NUM = 2000

NC = 2
NS = 16
CHUNK = 128
NCHT = 160
BLK = 80
EPT = NCHT * CHUNK
E_PAD = NS * EPT

NP = 10240
ROWS_PT = NP // NS
DN = 10240
DEG_PT = DN // NS

_mesh = plsc.VectorSubcoreMesh(core_axis_name="c", subcore_axis_name="s")


def _zero_rows(ref, rows, width):
    z = jnp.zeros((16,), jnp.float32)

    def body(i, _):
        for j in range(width // 16):
            ref[i, pl.ds(j * 16, 16)] = z
        return 0

    lax.fori_loop(0, rows, body, 0)


@functools.partial(
    pl.kernel,
    out_type=(
        jax.ShapeDtypeStruct(((K + 1) * NP, FH), jnp.float32),
        jax.ShapeDtypeStruct(((K + 1) * NP, FH), jnp.float32),
        jax.ShapeDtypeStruct((NC * DN,), jnp.float32),
    ),
    mesh=_mesh,
    compiler_params=pltpu.CompilerParams(use_tc_tiling_on_sc=False),
    scratch_types=[
        pltpu.VMEM((BLK, CHUNK), jnp.int32),
        pltpu.VMEM((BLK, CHUNK), jnp.int32),
        pltpu.VMEM((BLK, CHUNK), jnp.float32),
        pltpu.VMEM((CHUNK, FH), jnp.float32),
        pltpu.VMEM((CHUNK, FH), jnp.float32),
        pltpu.VMEM((CHUNK, FH), jnp.float32),
        pltpu.VMEM((CHUNK, FH), jnp.float32),
        pltpu.VMEM((CHUNK, FH), jnp.float32),
        pltpu.VMEM((DEG_PT,), jnp.float32),
        pltpu.SemaphoreType.DMA,
        pltpu.SemaphoreType.DMA,
        pltpu.SemaphoreType.DMA,
        pltpu.SemaphoreType.DMA,
        pltpu.SemaphoreType.DMA,
        pltpu.SemaphoreType.DMA,
        pltpu.SemaphoreType.DMA,
        pltpu.SemaphoreType.DMA,
        pltpu.VMEM_SHARED((NP, FH), jnp.float32),
        pltpu.VMEM_SHARED((DN,), jnp.float32),
    ],
)
def _diffuse_kernel(xl, xr, col3, row3, vals3, pl_out, pr_out, deg_out,
                    col_v, row_v, vals_v, g0, g1, g2, g3, zbuf, dz,
                    sg0, sg1, sg2, sg3, ss0, ss1, ss2, ss3, acc, dacc):
    c = lax.axis_index("c")
    s = lax.axis_index("s")

    _zero_rows(zbuf, CHUNK, FH)
    z16 = jnp.zeros((16,), jnp.float32)

    def dzb(i, _):
        dz[pl.ds(i * 16, 16)] = z16
        return 0

    lax.fori_loop(0, DEG_PT // 16, dzb, 0)

    pltpu.sync_copy(dz, dacc.at[pl.ds(s * DEG_PT, DEG_PT)])
    pltpu.sync_copy(row3.at[s, pl.ds(c * BLK, BLK)], row_v)
    pltpu.sync_copy(vals3.at[s, pl.ds(c * BLK, BLK)], vals_v)
    plsc.subcore_barrier()

    def deg_fire(i, _):
        pltpu.async_copy(vals_v.at[i], dacc.at[row_v.at[i]], sg0, add=True)
        return 0

    lax.fori_loop(0, BLK, deg_fire, 0)

    def deg_drain(i, _):
        pltpu.make_async_copy(vals_v.at[i], dacc.at[row_v.at[i]], sg0).wait()
        return 0

    lax.fori_loop(0, BLK, deg_drain, 0)
    plsc.subcore_barrier()
    pltpu.sync_copy(dacc.at[pl.ds(s * DEG_PT, DEG_PT)],
                    deg_out.at[pl.ds(c * DN + s * DEG_PT, DEG_PT)])

    def run_core(x_half, p_out):
        for i in range(ROWS_PT // CHUNK):
            pltpu.sync_copy(x_half.at[pl.ds(s * ROWS_PT + i * CHUNK, CHUNK)], g0)
            pltpu.sync_copy(g0, p_out.at[pl.ds(s * ROWS_PT + i * CHUNK, CHUNK)])
        plsc.subcore_barrier()

        def scale(buf, lc):
            def grp(g, _):
                vv = vals_v[lc, pl.ds(g * 16, 16)]
                base = g * 16
                for l in range(16):
                    v = vv[l]
                    for j in range(FH // 16):
                        buf[base + l, pl.ds(j * 16, 16)] = (
                            buf[base + l, pl.ds(j * 16, 16)] * v)
                return 0

            lax.fori_loop(0, CHUNK // 16, grp, 0)

        def round_body(k, _):
            for i in range(ROWS_PT // CHUNK):
                pltpu.sync_copy(zbuf, acc.at[pl.ds(s * ROWS_PT + i * CHUNK, CHUNK)])
            plsc.subcore_barrier()

            src_base = k * NP

            for b in range(NCHT // BLK):
                pltpu.sync_copy(col3.at[s, pl.ds(b * BLK, BLK)], col_v)
                pltpu.sync_copy(row3.at[s, pl.ds(b * BLK, BLK)], row_v)
                pltpu.sync_copy(vals3.at[s, pl.ds(b * BLK, BLK)], vals_v)


                def adj(i, _):
                    for j in range(CHUNK // 16):
                        col_v[i, pl.ds(j * 16, 16)] = (
                            col_v[i, pl.ds(j * 16, 16)] + src_base)
                    return 0

                lax.fori_loop(0, BLK, adj, 0)

                bufs = (g0, g1, g2, g3)
                sgs = (sg0, sg1, sg2, sg3)
                sss = (ss0, ss1, ss2, ss3)
                for b in range(4):
                    pltpu.async_copy(p_out.at[col_v.at[b]], bufs[b], sgs[b])

                def pipe(i, _):
                    for b in range(4):
                        lc = 4 * i + b
                        pltpu.make_async_copy(
                            p_out.at[col_v.at[lc]], bufs[b], sgs[b]).wait()
                        scale(bufs[b], lc)
                        pltpu.async_copy(
                            bufs[b], acc.at[row_v.at[lc]], sss[b], add=True)
                    for b in range(4):
                        lc = 4 * i + b
                        pltpu.make_async_copy(
                            bufs[b], acc.at[row_v.at[lc]], sss[b]).wait()

                        @pl.when(lc + 4 < BLK)
                        def _():
                            pltpu.async_copy(
                                p_out.at[col_v.at[lc + 4]], bufs[b], sgs[b])

                    return 0

                lax.fori_loop(0, BLK // 4, pipe, 0)

            plsc.subcore_barrier()
            dst = (k + 1) * NP + s * ROWS_PT
            pltpu.sync_copy(acc.at[pl.ds(s * ROWS_PT, ROWS_PT)],
                            p_out.at[pl.ds(dst, ROWS_PT)])
            plsc.subcore_barrier()
            return 0

        lax.fori_loop(0, K, round_body, 0)

    @pl.when(c == 0)
    def _():
        run_core(xl, pl_out)

    @pl.when(c == 1)
    def _():
        run_core(xr, pr_out)


_HB = 1024


def _softplus(z):
    return jnp.maximum(z, 0.0) + jnp.log1p(jnp.exp(-jnp.abs(z)))


def _hyper_body(x_ref, deg_ref, wp_ref, bp_ref, w1a_ref, w1b_ref, w1c_ref,
                b1_ref, w2_ref, b2_ref, sc_ref, coef_ref, alpha_ref, t_ref):
    pid = pl.program_id(0)
    x = x_ref[...]
    feat = jnp.dot(x, wp_ref[...], preferred_element_type=jnp.float32)
    feat = feat + bp_ref[...]

    d = deg_ref[0, :] + deg_ref[1, :]
    log_deg = jnp.log1p(d)[:, None]

    idx = pid * _HB + lax.broadcasted_iota(jnp.int32, (_HB, 1), 0)
    ntype = (idx >= DRUG_NUM).astype(jnp.float32)

    h = jnp.dot(feat, w1a_ref[...], preferred_element_type=jnp.float32)
    h = h + log_deg * w1b_ref[...] + ntype * w1c_ref[...] + b1_ref[...]
    h = jnp.maximum(h, 0.0)
    raw = jnp.dot(h, w2_ref[...], preferred_element_type=jnp.float32)
    raw = raw + b2_ref[...]

    sc = sc_ref[...]
    a_bias = jnp.where(idx < DRUG_NUM, sc[0, 0], sc[0, 1])
    alpha = _softplus(raw[:, 0:1] + a_bias) + 0.05
    t = _softplus(raw[:, 1:2] + sc[0, 2]) + 0.01
    alpha_ref[...] = alpha
    t_ref[...] = t

    a = jnp.clip(alpha, 0.05, 3.0)
    tt = jnp.clip(t, 0.01, 10.0)
    s = tt / (1.0 + tt)
    base = jnp.exp(-a * jnp.log1p(tt))
    cols = [base]
    rising = jnp.ones_like(a)
    s_pow = jnp.ones_like(a)
    for k in range(1, K + 1):
        rising = rising * (a + (k - 1.0)) / k
        s_pow = s_pow * s
        cols.append(base * rising * s_pow)
    coef_ref[...] = jnp.concatenate(cols, axis=1)


def _hyper_call(x, deg2, wp, bp, w1a, w1b, w1c, b1, w2, b2, scalars):
    full = lambda s: pl.BlockSpec(s, lambda i: tuple(0 for _ in s))
    return pl.pallas_call(
        _hyper_body,
        grid=(pl.cdiv(N, _HB),),
        in_specs=[
            pl.BlockSpec((_HB, F), lambda i: (i, 0)),
            pl.BlockSpec((NC, _HB), lambda i: (0, i)),
            full((F, F)), full((1, F)),
            full((F, F)), full((1, F)), full((1, F)), full((1, F)),
            full((F, F)), full((1, F)),
            full((1, 8)),
        ],
        out_specs=[
            pl.BlockSpec((_HB, K + 1), lambda i: (i, 0)),
            pl.BlockSpec((_HB, 1), lambda i: (i, 0)),
            pl.BlockSpec((_HB, 1), lambda i: (i, 0)),
        ],
        out_shape=[
            jax.ShapeDtypeStruct((N, K + 1), jnp.float32),
            jax.ShapeDtypeStruct((N, 1), jnp.float32),
            jax.ShapeDtypeStruct((N, 1), jnp.float32),
        ],
    )(x, deg2, wp, bp, w1a, w1b, w1c, b1, w2, b2, scalars)


def _wsum_body(coef_ref, pl_ref, pr_ref, out_ref):
    coef = coef_ref[...]
    acc_l = coef[:, 0:1] * pl_ref[0]
    acc_r = coef[:, 0:1] * pr_ref[0]
    for k in range(1, K + 1):
        acc_l = acc_l + coef[:, k:k + 1] * pl_ref[k]
        acc_r = acc_r + coef[:, k:k + 1] * pr_ref[k]
    out_ref[:, 0:FH] = acc_l
    out_ref[:, FH:F] = acc_r


def _wsum_call(coef, p_left, p_right):
    return pl.pallas_call(
        _wsum_body,
        grid=(pl.cdiv(N, _HB),),
        in_specs=[
            pl.BlockSpec((_HB, K + 1), lambda i: (i, 0)),
            pl.BlockSpec((K + 1, _HB, FH), lambda i: (0, i, 0)),
            pl.BlockSpec((K + 1, _HB, FH), lambda i: (0, i, 0)),
        ],
        out_specs=pl.BlockSpec((_HB, F), lambda i: (i, 0)),
        out_shape=jax.ShapeDtypeStruct((N, F), jnp.float32),
    )(coef, p_left, p_right)


def kernel(x, edge_index, edge_vals, W_proj, b_proj, W1, b1, W2, b2,
           alpha_bias_drug, alpha_bias_prot, t_bias):
    row = edge_index[0].astype(jnp.int32)
    col = edge_index[1].astype(jnp.int32)
    vals = edge_vals.astype(jnp.float32)
    order = jnp.argsort(col)
    row = row[order]
    col = col[order]
    vals = vals[order]

    pad = E_PAD - E
    zi = jnp.zeros((pad,), jnp.int32)
    row_p = jnp.concatenate([row, zi]).reshape(NS, NCHT, CHUNK)
    col_p = jnp.concatenate([col, zi]).reshape(NS, NCHT, CHUNK)
    vals_p = jnp.concatenate([vals, jnp.zeros((pad,), jnp.float32)]
                             ).reshape(NS, NCHT, CHUNK)

    x_pad = jnp.zeros((NP, F), jnp.float32).at[:N].set(x)
    xl = x_pad[:, :FH]
    xr = x_pad[:, FH:]

    p_left, p_right, deg = _diffuse_kernel(xl, xr, col_p, row_p, vals_p)
    deg2 = deg.reshape(NC, DN)

    wp = jnp.zeros((F, F), jnp.float32).at[:, :W_proj.shape[0]].set(W_proj.T)
    bp = jnp.zeros((1, F), jnp.float32).at[0, :b_proj.shape[0]].set(b_proj)
    nh = W1.shape[0]
    w1a = jnp.zeros((F, F), jnp.float32).at[:W1.shape[1] - 2, :nh].set(W1[:, :-2].T)
    w1b = jnp.zeros((1, F), jnp.float32).at[0, :nh].set(W1[:, -2])
    w1c = jnp.zeros((1, F), jnp.float32).at[0, :nh].set(W1[:, -1])
    b1p = jnp.zeros((1, F), jnp.float32).at[0, :nh].set(b1)
    w2 = jnp.zeros((F, F), jnp.float32).at[:nh, :2].set(W2.T)
    b2p = jnp.zeros((1, F), jnp.float32).at[0, :2].set(b2)
    scalars = jnp.zeros((1, 8), jnp.float32)
    scalars = scalars.at[0, 0].set(alpha_bias_drug)
    scalars = scalars.at[0, 1].set(alpha_bias_prot)
    scalars = scalars.at[0, 2].set(t_bias)

    coef, alpha2, t2 = _hyper_call(x, deg2, wp, bp, w1a, w1b, w1c, b1p,
                                   w2, b2p, scalars)

    out = _wsum_call(coef,
                     p_left.reshape(K + 1, NP, FH),
                     p_right.reshape(K + 1, NP, FH))
    return (out, alpha2[:, 0], t2[:, 0])


# speedup vs baseline: 1.2872x; 1.2872x over previous
import functools

import jax
import jax.numpy as jnp
from jax import lax
from jax.experimental import pallas as pl
from jax.experimental.pallas import tpu as pltpu
from jax.experimental.pallas import tpu_sc as plsc

N = 10000
F = 128
FH = F // 2
E = 320000
K = 15
DRUG_NUM = 2000

NC = 2
NS = 16
CHUNK = 128
NCHT = 160
BLK = 80
EPT = NCHT * CHUNK
E_PAD = NS * EPT

NP = 10240
ROWS_PT = NP // NS
DN = 10240
DEG_PT = DN // NS

_mesh = plsc.VectorSubcoreMesh(core_axis_name="c", subcore_axis_name="s")


def _zero_rows(ref, rows, width):
    z = jnp.zeros((16,), jnp.float32)

    def body(i, _):
        for j in range(width // 16):
            ref[i, pl.ds(j * 16, 16)] = z
        return 0

    lax.fori_loop(0, rows, body, 0)


@functools.partial(
    pl.kernel,
    out_type=(
        jax.ShapeDtypeStruct(((K + 1) * NP, FH), jnp.float32),
        jax.ShapeDtypeStruct(((K + 1) * NP, FH), jnp.float32),
        jax.ShapeDtypeStruct((NC * DN,), jnp.float32),
    ),
    mesh=_mesh,
    compiler_params=pltpu.CompilerParams(use_tc_tiling_on_sc=False),
    scratch_types=[
        pltpu.VMEM((BLK, CHUNK), jnp.int32),
        pltpu.VMEM((BLK, CHUNK), jnp.int32),
        pltpu.VMEM((BLK, CHUNK), jnp.float32),
        pltpu.VMEM((CHUNK, FH), jnp.float32),
        pltpu.VMEM((CHUNK, FH), jnp.float32),
        pltpu.VMEM((CHUNK, FH), jnp.float32),
        pltpu.VMEM((CHUNK, FH), jnp.float32),
        pltpu.VMEM((CHUNK, 16), jnp.float32),
        pltpu.VMEM((CHUNK, 16), jnp.float32),
        pltpu.VMEM((CHUNK, 16), jnp.float32),
        pltpu.VMEM((CHUNK, 16), jnp.float32),
        pltpu.VMEM((CHUNK, FH), jnp.float32),
        pltpu.VMEM((DEG_PT,), jnp.float32),
        pltpu.SemaphoreType.DMA,
        pltpu.SemaphoreType.DMA,
        pltpu.SemaphoreType.DMA,
        pltpu.SemaphoreType.DMA,
        pltpu.SemaphoreType.DMA,
        pltpu.SemaphoreType.DMA,
        pltpu.SemaphoreType.DMA,
        pltpu.SemaphoreType.DMA,
        pltpu.VMEM_SHARED((NP, FH), jnp.float32),
        pltpu.VMEM_SHARED((DN,), jnp.float32),
    ],
)
def _diffuse_kernel(xl, xr, col3, row3, vals3, vx3, pl_out, pr_out, deg_out,
                    col_v, row_v, vals_v, g0, g1, g2, g3,
                    vx0, vx1, vx2, vx3b, zbuf, dz,
                    sg0, sg1, sg2, sg3, ss0, ss1, ss2, ss3, acc, dacc):
    c = lax.axis_index("c")
    s = lax.axis_index("s")

    _zero_rows(zbuf, CHUNK, FH)
    z16 = jnp.zeros((16,), jnp.float32)

    def dzb(i, _):
        dz[pl.ds(i * 16, 16)] = z16
        return 0

    lax.fori_loop(0, DEG_PT // 16, dzb, 0)

    pltpu.sync_copy(dz, dacc.at[pl.ds(s * DEG_PT, DEG_PT)])
    pltpu.sync_copy(row3.at[s, pl.ds(c * BLK, BLK)], row_v)
    pltpu.sync_copy(vals3.at[s, pl.ds(c * BLK, BLK)], vals_v)
    plsc.subcore_barrier()

    def deg_fire(i, _):
        pltpu.async_copy(vals_v.at[i], dacc.at[row_v.at[i]], sg0, add=True)
        return 0

    lax.fori_loop(0, BLK, deg_fire, 0)

    def deg_drain(i, _):
        pltpu.make_async_copy(vals_v.at[i], dacc.at[row_v.at[i]], sg0).wait()
        return 0

    lax.fori_loop(0, BLK, deg_drain, 0)
    plsc.subcore_barrier()
    pltpu.sync_copy(dacc.at[pl.ds(s * DEG_PT, DEG_PT)],
                    deg_out.at[pl.ds(c * DN + s * DEG_PT, DEG_PT)])

    def run_core(x_half, p_out):
        for i in range(ROWS_PT // CHUNK):
            pltpu.sync_copy(x_half.at[pl.ds(s * ROWS_PT + i * CHUNK, CHUNK)], g0)
            pltpu.sync_copy(g0, p_out.at[pl.ds(s * ROWS_PT + i * CHUNK, CHUNK)])
        plsc.subcore_barrier()

        def scale(buf, vxbuf):
            def grp(g, _):
                base = g * 16
                for l in range(16):
                    vv = vxbuf[base + l, :]
                    for j in range(FH // 16):
                        buf[base + l, pl.ds(j * 16, 16)] = (
                            buf[base + l, pl.ds(j * 16, 16)] * vv)
                return 0

            lax.fori_loop(0, CHUNK // 16, grp, 0)

        def round_body(k, _):
            for i in range(ROWS_PT // CHUNK):
                pltpu.sync_copy(zbuf, acc.at[pl.ds(s * ROWS_PT + i * CHUNK, CHUNK)])
            plsc.subcore_barrier()

            src_base = k * NP

            for b in range(NCHT // BLK):
                pltpu.sync_copy(col3.at[s, pl.ds(b * BLK, BLK)], col_v)
                pltpu.sync_copy(row3.at[s, pl.ds(b * BLK, BLK)], row_v)
                pltpu.sync_copy(vals3.at[s, pl.ds(b * BLK, BLK)], vals_v)


                def adj(i, _):
                    for j in range(CHUNK // 16):
                        col_v[i, pl.ds(j * 16, 16)] = (
                            col_v[i, pl.ds(j * 16, 16)] + src_base)
                    return 0

                lax.fori_loop(0, BLK, adj, 0)

                bufs = (g0, g1, g2, g3)
                vxb = (vx0, vx1, vx2, vx3b)
                sgs = (sg0, sg1, sg2, sg3)
                sss = (ss0, ss1, ss2, ss3)
                bb = b
                for q in range(4):
                    pltpu.async_copy(p_out.at[col_v.at[q]], bufs[q], sgs[q])
                    pltpu.async_copy(vx3.at[s, bb * BLK + q], vxb[q], sgs[q])

                def pipe(i, _):
                    for q in range(4):
                        lc = 4 * i + q
                        pltpu.make_async_copy(
                            p_out.at[col_v.at[lc]], bufs[q], sgs[q]).wait()
                        pltpu.make_async_copy(
                            vx3.at[s, bb * BLK + lc], vxb[q], sgs[q]).wait()
                        scale(bufs[q], vxb[q])
                        pltpu.async_copy(
                            bufs[q], acc.at[row_v.at[lc]], sss[q], add=True)
                    for q in range(4):
                        lc = 4 * i + q
                        pltpu.make_async_copy(
                            bufs[q], acc.at[row_v.at[lc]], sss[q]).wait()

                        @pl.when(lc + 4 < BLK)
                        def _():
                            pltpu.async_copy(
                                p_out.at[col_v.at[lc + 4]], bufs[q], sgs[q])
                            pltpu.async_copy(
                                vx3.at[s, bb * BLK + lc + 4], vxb[q], sgs[q])

                    return 0

                lax.fori_loop(0, BLK // 4, pipe, 0)

            plsc.subcore_barrier()
            dst = (k + 1) * NP + s * ROWS_PT
            pltpu.sync_copy(acc.at[pl.ds(s * ROWS_PT, ROWS_PT)],
                            p_out.at[pl.ds(dst, ROWS_PT)])
            plsc.subcore_barrier()
            return 0

        lax.fori_loop(0, K, round_body, 0)

    @pl.when(c == 0)
    def _():
        run_core(xl, pl_out)

    @pl.when(c == 1)
    def _():
        run_core(xr, pr_out)


_HB = 1024


def _softplus(z):
    return jnp.maximum(z, 0.0) + jnp.log1p(jnp.exp(-jnp.abs(z)))


def _hyper_body(x_ref, deg_ref, wp_ref, bp_ref, w1a_ref, w1b_ref, w1c_ref,
                b1_ref, w2_ref, b2_ref, sc_ref, coef_ref, alpha_ref, t_ref):
    pid = pl.program_id(0)
    x = x_ref[...]
    feat = jnp.dot(x, wp_ref[...], preferred_element_type=jnp.float32)
    feat = feat + bp_ref[...]

    d = deg_ref[0, :] + deg_ref[1, :]
    log_deg = jnp.log1p(d)[:, None]

    idx = pid * _HB + lax.broadcasted_iota(jnp.int32, (_HB, 1), 0)
    ntype = (idx >= DRUG_NUM).astype(jnp.float32)

    h = jnp.dot(feat, w1a_ref[...], preferred_element_type=jnp.float32)
    h = h + log_deg * w1b_ref[...] + ntype * w1c_ref[...] + b1_ref[...]
    h = jnp.maximum(h, 0.0)
    raw = jnp.dot(h, w2_ref[...], preferred_element_type=jnp.float32)
    raw = raw + b2_ref[...]

    sc = sc_ref[...]
    a_bias = jnp.where(idx < DRUG_NUM, sc[0, 0], sc[0, 1])
    alpha = _softplus(raw[:, 0:1] + a_bias) + 0.05
    t = _softplus(raw[:, 1:2] + sc[0, 2]) + 0.01
    alpha_ref[...] = alpha
    t_ref[...] = t

    a = jnp.clip(alpha, 0.05, 3.0)
    tt = jnp.clip(t, 0.01, 10.0)
    s = tt / (1.0 + tt)
    base = jnp.exp(-a * jnp.log1p(tt))
    cols = [base]
    rising = jnp.ones_like(a)
    s_pow = jnp.ones_like(a)
    for k in range(1, K + 1):
        rising = rising * (a + (k - 1.0)) / k
        s_pow = s_pow * s
        cols.append(base * rising * s_pow)
    coef_ref[...] = jnp.concatenate(cols, axis=1)


def _hyper_call(x, deg2, wp, bp, w1a, w1b, w1c, b1, w2, b2, scalars):
    full = lambda s: pl.BlockSpec(s, lambda i: tuple(0 for _ in s))
    return pl.pallas_call(
        _hyper_body,
        grid=(pl.cdiv(N, _HB),),
        in_specs=[
            pl.BlockSpec((_HB, F), lambda i: (i, 0)),
            pl.BlockSpec((NC, _HB), lambda i: (0, i)),
            full((F, F)), full((1, F)),
            full((F, F)), full((1, F)), full((1, F)), full((1, F)),
            full((F, F)), full((1, F)),
            full((1, 8)),
        ],
        out_specs=[
            pl.BlockSpec((_HB, K + 1), lambda i: (i, 0)),
            pl.BlockSpec((_HB, 1), lambda i: (i, 0)),
            pl.BlockSpec((_HB, 1), lambda i: (i, 0)),
        ],
        out_shape=[
            jax.ShapeDtypeStruct((N, K + 1), jnp.float32),
            jax.ShapeDtypeStruct((N, 1), jnp.float32),
            jax.ShapeDtypeStruct((N, 1), jnp.float32),
        ],
    )(x, deg2, wp, bp, w1a, w1b, w1c, b1, w2, b2, scalars)


def _wsum_body(coef_ref, pl_ref, pr_ref, out_ref):
    coef = coef_ref[...]
    acc_l = coef[:, 0:1] * pl_ref[0]
    acc_r = coef[:, 0:1] * pr_ref[0]
    for k in range(1, K + 1):
        acc_l = acc_l + coef[:, k:k + 1] * pl_ref[k]
        acc_r = acc_r + coef[:, k:k + 1] * pr_ref[k]
    out_ref[:, 0:FH] = acc_l
    out_ref[:, FH:F] = acc_r


def _wsum_call(coef, p_left, p_right):
    return pl.pallas_call(
        _wsum_body,
        grid=(pl.cdiv(N, _HB),),
        in_specs=[
            pl.BlockSpec((_HB, K + 1), lambda i: (i, 0)),
            pl.BlockSpec((K + 1, _HB, FH), lambda i: (0, i, 0)),
            pl.BlockSpec((K + 1, _HB, FH), lambda i: (0, i, 0)),
        ],
        out_specs=pl.BlockSpec((_HB, F), lambda i: (i, 0)),
        out_shape=jax.ShapeDtypeStruct((N, F), jnp.float32),
    )(coef, p_left, p_right)


def kernel(x, edge_index, edge_vals, W_proj, b_proj, W1, b1, W2, b2,
           alpha_bias_drug, alpha_bias_prot, t_bias):
    row = edge_index[0].astype(jnp.int32)
    col = edge_index[1].astype(jnp.int32)
    vals = edge_vals.astype(jnp.float32)

    pad = E_PAD - E
    zi = jnp.zeros((pad,), jnp.int32)
    row_p = jnp.concatenate([row, zi]).reshape(NS, NCHT, CHUNK)
    col_p = jnp.concatenate([col, zi]).reshape(NS, NCHT, CHUNK)
    vals_p = jnp.concatenate([vals, jnp.zeros((pad,), jnp.float32)]
                             ).reshape(NS, NCHT, CHUNK)

    x_pad = jnp.zeros((NP, F), jnp.float32).at[:N].set(x)
    xl = x_pad[:, :FH]
    xr = x_pad[:, FH:]

    vx = jnp.broadcast_to(vals_p[..., None], (NS, NCHT, CHUNK, 16))
    vx = vx + jnp.zeros((16,), jnp.float32)
    p_left, p_right, deg = _diffuse_kernel(xl, xr, col_p, row_p, vals_p, vx)
    deg2 = deg.reshape(NC, DN)

    wp = jnp.zeros((F, F), jnp.float32).at[:, :W_proj.shape[0]].set(W_proj.T)
    bp = jnp.zeros((1, F), jnp.float32).at[0, :b_proj.shape[0]].set(b_proj)
    nh = W1.shape[0]
    w1a = jnp.zeros((F, F), jnp.float32).at[:W1.shape[1] - 2, :nh].set(W1[:, :-2].T)
    w1b = jnp.zeros((1, F), jnp.float32).at[0, :nh].set(W1[:, -2])
    w1c = jnp.zeros((1, F), jnp.float32).at[0, :nh].set(W1[:, -1])
    b1p = jnp.zeros((1, F), jnp.float32).at[0, :nh].set(b1)
    w2 = jnp.zeros((F, F), jnp.float32).at[:nh, :2].set(W2.T)
    b2p = jnp.zeros((1, F), jnp.float32).at[0, :2].set(b2)
    scalars = jnp.zeros((1, 8), jnp.float32)
    scalars = scalars.at[0, 0].set(alpha_bias_drug)
    scalars = scalars.at[0, 1].set(alpha_bias_prot)
    scalars = scalars.at[0, 2].set(t_bias)

    coef, alpha2, t2 = _hyper_call(x, deg2, wp, bp, w1a, w1b, w1c, b1p,
                                   w2, b2p, scalars)

    out = _wsum_call(coef,
                     p_left.reshape(K + 1, NP, FH),
                     p_right.reshape(K + 1, NP, FH))
    return (out, alpha2[:, 0], t2[:, 0])


# speedup vs baseline: 1.2892x; 1.0015x over previous
import functools

import jax
import jax.numpy as jnp
from jax import lax
from jax.experimental import pallas as pl
from jax.experimental.pallas import tpu as pltpu
from jax.experimental.pallas import tpu_sc as plsc

N = 10000
F = 128
FH = F // 2
E = 320000
K = 15
DRUG_NUM = 2000

NC = 2
NS = 16
CHUNK = 128
NCHT = 160
BLK = 80
EPT = NCHT * CHUNK
E_PAD = NS * EPT

NP = 10240
ROWS_PT = NP // NS
DN = 10240
DEG_PT = DN // NS

_mesh = plsc.VectorSubcoreMesh(core_axis_name="c", subcore_axis_name="s")


def _zero_rows(ref, rows, width):
    z = jnp.zeros((16,), jnp.float32)

    def body(i, _):
        for j in range(width // 16):
            ref[i, pl.ds(j * 16, 16)] = z
        return 0

    lax.fori_loop(0, rows, body, 0)


@functools.partial(
    pl.kernel,
    out_type=(
        jax.ShapeDtypeStruct(((K + 1) * NP, FH), jnp.float32),
        jax.ShapeDtypeStruct(((K + 1) * NP, FH), jnp.float32),
        jax.ShapeDtypeStruct((NC * DN,), jnp.float32),
    ),
    mesh=_mesh,
    compiler_params=pltpu.CompilerParams(use_tc_tiling_on_sc=False),
    scratch_types=[
        pltpu.VMEM((BLK, CHUNK), jnp.int32),
        pltpu.VMEM((BLK, CHUNK), jnp.int32),
        pltpu.VMEM((BLK, CHUNK), jnp.float32),
        pltpu.VMEM((CHUNK, FH), jnp.float32),
        pltpu.VMEM((CHUNK, FH), jnp.float32),
        pltpu.VMEM((CHUNK, FH), jnp.float32),
        pltpu.VMEM((CHUNK, FH), jnp.float32),
        pltpu.VMEM((CHUNK, 16), jnp.float32),
        pltpu.VMEM((CHUNK, 16), jnp.float32),
        pltpu.VMEM((CHUNK, 16), jnp.float32),
        pltpu.VMEM((CHUNK, 16), jnp.float32),
        pltpu.VMEM((CHUNK, FH), jnp.float32),
        pltpu.VMEM((DEG_PT,), jnp.float32),
        pltpu.SemaphoreType.DMA,
        pltpu.SemaphoreType.DMA,
        pltpu.SemaphoreType.DMA,
        pltpu.SemaphoreType.DMA,
        pltpu.SemaphoreType.DMA,
        pltpu.SemaphoreType.DMA,
        pltpu.SemaphoreType.DMA,
        pltpu.SemaphoreType.DMA,
        pltpu.VMEM_SHARED((NP, FH), jnp.float32),
        pltpu.VMEM_SHARED((DN,), jnp.float32),
    ],
)
def _diffuse_kernel(xl, xr, col3, row3, vals3, vx3, pl_out, pr_out, deg_out,
                    col_v, row_v, vals_v, g0, g1, g2, g3,
                    vx0, vx1, vx2, vx3b, zbuf, dz,
                    sg0, sg1, sg2, sg3, ss0, ss1, ss2, ss3, acc, dacc):
    c = lax.axis_index("c")
    s = lax.axis_index("s")

    _zero_rows(zbuf, CHUNK, FH)
    z16 = jnp.zeros((16,), jnp.float32)

    def dzb(i, _):
        dz[pl.ds(i * 16, 16)] = z16
        return 0

    lax.fori_loop(0, DEG_PT // 16, dzb, 0)

    pltpu.sync_copy(dz, dacc.at[pl.ds(s * DEG_PT, DEG_PT)])
    pltpu.sync_copy(row3.at[s, pl.ds(c * BLK, BLK)], row_v)
    pltpu.sync_copy(vals3.at[s, pl.ds(c * BLK, BLK)], vals_v)
    plsc.subcore_barrier()

    def deg_fire(i, _):
        pltpu.async_copy(vals_v.at[i], dacc.at[row_v.at[i]], sg0, add=True)
        return 0

    lax.fori_loop(0, BLK, deg_fire, 0)

    def deg_drain(i, _):
        pltpu.make_async_copy(vals_v.at[i], dacc.at[row_v.at[i]], sg0).wait()
        return 0

    lax.fori_loop(0, BLK, deg_drain, 0)
    plsc.subcore_barrier()
    pltpu.sync_copy(dacc.at[pl.ds(s * DEG_PT, DEG_PT)],
                    deg_out.at[pl.ds(c * DN + s * DEG_PT, DEG_PT)])

    def src_base0(k):
        return k * NP

    def run_core(x_half, p_out):
        for i in range(ROWS_PT // CHUNK):
            pltpu.sync_copy(x_half.at[pl.ds(s * ROWS_PT + i * CHUNK, CHUNK)], g0)
            pltpu.sync_copy(g0, p_out.at[pl.ds(s * ROWS_PT + i * CHUNK, CHUNK)])
        plsc.subcore_barrier()

        def scale(buf, vxbuf):
            def grp(g, _):
                base = g * 16
                for l in range(16):
                    vv = vxbuf[base + l, :]
                    for j in range(FH // 16):
                        buf[base + l, pl.ds(j * 16, 16)] = (
                            buf[base + l, pl.ds(j * 16, 16)] * vv)
                return 0

            lax.fori_loop(0, CHUNK // 16, grp, 0)

        def round_body(k, _):
            p_src = p_out.at[pl.ds(src_base0(k), NP)]
            for i in range(ROWS_PT // CHUNK):
                pltpu.sync_copy(zbuf, acc.at[pl.ds(s * ROWS_PT + i * CHUNK, CHUNK)])
            plsc.subcore_barrier()


            for b in range(NCHT // BLK):
                pltpu.sync_copy(col3.at[s, pl.ds(b * BLK, BLK)], col_v)
                pltpu.sync_copy(row3.at[s, pl.ds(b * BLK, BLK)], row_v)
                pltpu.sync_copy(vals3.at[s, pl.ds(b * BLK, BLK)], vals_v)


                bufs = (g0, g1, g2, g3)
                vxb = (vx0, vx1, vx2, vx3b)
                sgs = (sg0, sg1, sg2, sg3)
                sss = (ss0, ss1, ss2, ss3)
                bb = b
                for q in range(4):
                    pltpu.async_copy(p_src.at[col_v.at[q]], bufs[q], sgs[q])
                    pltpu.async_copy(vx3.at[s, bb * BLK + q], vxb[q], sgs[q])

                def pipe(i, _):
                    for q in range(4):
                        lc = 4 * i + q
                        pltpu.make_async_copy(
                            p_src.at[col_v.at[lc]], bufs[q], sgs[q]).wait()
                        pltpu.make_async_copy(
                            vx3.at[s, bb * BLK + lc], vxb[q], sgs[q]).wait()
                        scale(bufs[q], vxb[q])
                        pltpu.async_copy(
                            bufs[q], acc.at[row_v.at[lc]], sss[q], add=True)
                    for q in range(4):
                        lc = 4 * i + q
                        pltpu.make_async_copy(
                            bufs[q], acc.at[row_v.at[lc]], sss[q]).wait()

                        @pl.when(lc + 4 < BLK)
                        def _():
                            pltpu.async_copy(
                                p_src.at[col_v.at[lc + 4]], bufs[q], sgs[q])
                            pltpu.async_copy(
                                vx3.at[s, bb * BLK + lc + 4], vxb[q], sgs[q])

                    return 0

                lax.fori_loop(0, BLK // 4, pipe, 0)

            plsc.subcore_barrier()
            dst = (k + 1) * NP + s * ROWS_PT
            pltpu.sync_copy(acc.at[pl.ds(s * ROWS_PT, ROWS_PT)],
                            p_out.at[pl.ds(dst, ROWS_PT)])
            plsc.subcore_barrier()
            return 0

        lax.fori_loop(0, K, round_body, 0)

    @pl.when(c == 0)
    def _():
        run_core(xl, pl_out)

    @pl.when(c == 1)
    def _():
        run_core(xr, pr_out)


_HB = 1024


def _softplus(z):
    return jnp.maximum(z, 0.0) + jnp.log1p(jnp.exp(-jnp.abs(z)))


def _hyper_body(x_ref, deg_ref, wp_ref, bp_ref, w1a_ref, w1b_ref, w1c_ref,
                b1_ref, w2_ref, b2_ref, sc_ref, coef_ref, alpha_ref, t_ref):
    pid = pl.program_id(0)
    x = x_ref[...]
    feat = jnp.dot(x, wp_ref[...], preferred_element_type=jnp.float32)
    feat = feat + bp_ref[...]

    d = deg_ref[0, :] + deg_ref[1, :]
    log_deg = jnp.log1p(d)[:, None]

    idx = pid * _HB + lax.broadcasted_iota(jnp.int32, (_HB, 1), 0)
    ntype = (idx >= DRUG_NUM).astype(jnp.float32)

    h = jnp.dot(feat, w1a_ref[...], preferred_element_type=jnp.float32)
    h = h + log_deg * w1b_ref[...] + ntype * w1c_ref[...] + b1_ref[...]
    h = jnp.maximum(h, 0.0)
    raw = jnp.dot(h, w2_ref[...], preferred_element_type=jnp.float32)
    raw = raw + b2_ref[...]

    sc = sc_ref[...]
    a_bias = jnp.where(idx < DRUG_NUM, sc[0, 0], sc[0, 1])
    alpha = _softplus(raw[:, 0:1] + a_bias) + 0.05
    t = _softplus(raw[:, 1:2] + sc[0, 2]) + 0.01
    alpha_ref[...] = alpha
    t_ref[...] = t

    a = jnp.clip(alpha, 0.05, 3.0)
    tt = jnp.clip(t, 0.01, 10.0)
    s = tt / (1.0 + tt)
    base = jnp.exp(-a * jnp.log1p(tt))
    cols = [base]
    rising = jnp.ones_like(a)
    s_pow = jnp.ones_like(a)
    for k in range(1, K + 1):
        rising = rising * (a + (k - 1.0)) / k
        s_pow = s_pow * s
        cols.append(base * rising * s_pow)
    coef_ref[...] = jnp.concatenate(cols, axis=1)


def _hyper_call(x, deg2, wp, bp, w1a, w1b, w1c, b1, w2, b2, scalars):
    full = lambda s: pl.BlockSpec(s, lambda i: tuple(0 for _ in s))
    return pl.pallas_call(
        _hyper_body,
        grid=(pl.cdiv(N, _HB),),
        in_specs=[
            pl.BlockSpec((_HB, F), lambda i: (i, 0)),
            pl.BlockSpec((NC, _HB), lambda i: (0, i)),
            full((F, F)), full((1, F)),
            full((F, F)), full((1, F)), full((1, F)), full((1, F)),
            full((F, F)), full((1, F)),
            full((1, 8)),
        ],
        out_specs=[
            pl.BlockSpec((_HB, K + 1), lambda i: (i, 0)),
            pl.BlockSpec((_HB, 1), lambda i: (i, 0)),
            pl.BlockSpec((_HB, 1), lambda i: (i, 0)),
        ],
        out_shape=[
            jax.ShapeDtypeStruct((N, K + 1), jnp.float32),
            jax.ShapeDtypeStruct((N, 1), jnp.float32),
            jax.ShapeDtypeStruct((N, 1), jnp.float32),
        ],
    )(x, deg2, wp, bp, w1a, w1b, w1c, b1, w2, b2, scalars)


def _wsum_body(coef_ref, pl_ref, pr_ref, out_ref):
    coef = coef_ref[...]
    acc_l = coef[:, 0:1] * pl_ref[0]
    acc_r = coef[:, 0:1] * pr_ref[0]
    for k in range(1, K + 1):
        acc_l = acc_l + coef[:, k:k + 1] * pl_ref[k]
        acc_r = acc_r + coef[:, k:k + 1] * pr_ref[k]
    out_ref[:, 0:FH] = acc_l
    out_ref[:, FH:F] = acc_r


def _wsum_call(coef, p_left, p_right):
    return pl.pallas_call(
        _wsum_body,
        grid=(pl.cdiv(N, _HB),),
        in_specs=[
            pl.BlockSpec((_HB, K + 1), lambda i: (i, 0)),
            pl.BlockSpec((K + 1, _HB, FH), lambda i: (0, i, 0)),
            pl.BlockSpec((K + 1, _HB, FH), lambda i: (0, i, 0)),
        ],
        out_specs=pl.BlockSpec((_HB, F), lambda i: (i, 0)),
        out_shape=jax.ShapeDtypeStruct((N, F), jnp.float32),
    )(coef, p_left, p_right)


def kernel(x, edge_index, edge_vals, W_proj, b_proj, W1, b1, W2, b2,
           alpha_bias_drug, alpha_bias_prot, t_bias):
    row = edge_index[0].astype(jnp.int32)
    col = edge_index[1].astype(jnp.int32)
    vals = edge_vals.astype(jnp.float32)

    pad = E_PAD - E
    zi = jnp.zeros((pad,), jnp.int32)
    row_p = jnp.concatenate([row, zi]).reshape(NS, NCHT, CHUNK)
    col_p = jnp.concatenate([col, zi]).reshape(NS, NCHT, CHUNK)
    vals_p = jnp.concatenate([vals, jnp.zeros((pad,), jnp.float32)]
                             ).reshape(NS, NCHT, CHUNK)

    x_pad = jnp.zeros((NP, F), jnp.float32).at[:N].set(x)
    xl = x_pad[:, :FH]
    xr = x_pad[:, FH:]

    vx = jnp.broadcast_to(vals_p[..., None], (NS, NCHT, CHUNK, 16))
    vx = vx + jnp.zeros((16,), jnp.float32)
    p_left, p_right, deg = _diffuse_kernel(xl, xr, col_p, row_p, vals_p, vx)
    deg2 = deg.reshape(NC, DN)

    wp = jnp.zeros((F, F), jnp.float32).at[:, :W_proj.shape[0]].set(W_proj.T)
    bp = jnp.zeros((1, F), jnp.float32).at[0, :b_proj.shape[0]].set(b_proj)
    nh = W1.shape[0]
    w1a = jnp.zeros((F, F), jnp.float32).at[:W1.shape[1] - 2, :nh].set(W1[:, :-2].T)
    w1b = jnp.zeros((1, F), jnp.float32).at[0, :nh].set(W1[:, -2])
    w1c = jnp.zeros((1, F), jnp.float32).at[0, :nh].set(W1[:, -1])
    b1p = jnp.zeros((1, F), jnp.float32).at[0, :nh].set(b1)
    w2 = jnp.zeros((F, F), jnp.float32).at[:nh, :2].set(W2.T)
    b2p = jnp.zeros((1, F), jnp.float32).at[0, :2].set(b2)
    scalars = jnp.zeros((1, 8), jnp.float32)
    scalars = scalars.at[0, 0].set(alpha_bias_drug)
    scalars = scalars.at[0, 1].set(alpha_bias_prot)
    scalars = scalars.at[0, 2].set(t_bias)

    coef, alpha2, t2 = _hyper_call(x, deg2, wp, bp, w1a, w1b, w1c, b1p,
                                   w2, b2p, scalars)

    out = _wsum_call(coef,
                     p_left.reshape(K + 1, NP, FH),
                     p_right.reshape(K + 1, NP, FH))
    return (out, alpha2[:, 0], t2[:, 0])


# speedup vs baseline: 1.2894x; 1.0002x over previous
import functools

import jax
import jax.numpy as jnp
from jax import lax
from jax.experimental import pallas as pl
from jax.experimental.pallas import tpu as pltpu
from jax.experimental.pallas import tpu_sc as plsc

N = 10000
F = 128
FH = F // 2
E = 320000
K = 15
DRUG_NUM = 2000

NC = 2
NS = 16
CHUNK = 128
NCHT = 160
BLK = 80
EPT = NCHT * CHUNK
E_PAD = NS * EPT

NP = 10240
ROWS_PT = NP // NS
DN = 10240
DEG_PT = DN // NS

_mesh = plsc.VectorSubcoreMesh(core_axis_name="c", subcore_axis_name="s")


def _zero_rows(ref, rows, width):
    z = jnp.zeros((16,), jnp.float32)

    def body(i, _):
        for j in range(width // 16):
            ref[i, pl.ds(j * 16, 16)] = z
        return 0

    lax.fori_loop(0, rows, body, 0)


@functools.partial(
    pl.kernel,
    out_type=(
        jax.ShapeDtypeStruct(((K + 1) * NP, FH), jnp.float32),
        jax.ShapeDtypeStruct(((K + 1) * NP, FH), jnp.float32),
        jax.ShapeDtypeStruct((NC * DN,), jnp.float32),
    ),
    mesh=_mesh,
    compiler_params=pltpu.CompilerParams(use_tc_tiling_on_sc=False),
    scratch_types=[
        pltpu.VMEM((BLK, CHUNK), jnp.int32),
        pltpu.VMEM((BLK, CHUNK), jnp.int32),
        pltpu.VMEM((BLK, CHUNK), jnp.float32),
        pltpu.VMEM((CHUNK, FH), jnp.float32),
        pltpu.VMEM((CHUNK, FH), jnp.float32),
        pltpu.VMEM((CHUNK, FH), jnp.float32),
        pltpu.VMEM((CHUNK, FH), jnp.float32),
        pltpu.VMEM((CHUNK, 16), jnp.float32),
        pltpu.VMEM((CHUNK, 16), jnp.float32),
        pltpu.VMEM((CHUNK, 16), jnp.float32),
        pltpu.VMEM((CHUNK, 16), jnp.float32),
        pltpu.VMEM((CHUNK, FH), jnp.float32),
        pltpu.VMEM((DEG_PT,), jnp.float32),
        pltpu.SemaphoreType.DMA,
        pltpu.SemaphoreType.DMA,
        pltpu.SemaphoreType.DMA,
        pltpu.SemaphoreType.DMA,
        pltpu.SemaphoreType.DMA,
        pltpu.SemaphoreType.DMA,
        pltpu.SemaphoreType.DMA,
        pltpu.SemaphoreType.DMA,
        pltpu.VMEM_SHARED((NP, FH), jnp.float32),
        pltpu.VMEM_SHARED((DN,), jnp.float32),
    ],
)
def _diffuse_kernel(xl, xr, col3, row3, vals3, vx3, pl_out, pr_out, deg_out,
                    col_v, row_v, vals_v, g0, g1, g2, g3,
                    vx0, vx1, vx2, vx3b, zbuf, dz,
                    sg0, sg1, sg2, sg3, ss0, ss1, ss2, ss3, acc, dacc):
    c = lax.axis_index("c")
    s = lax.axis_index("s")

    _zero_rows(zbuf, CHUNK, FH)
    z16 = jnp.zeros((16,), jnp.float32)

    def dzb(i, _):
        dz[pl.ds(i * 16, 16)] = z16
        return 0

    lax.fori_loop(0, DEG_PT // 16, dzb, 0)

    pltpu.sync_copy(dz, dacc.at[pl.ds(s * DEG_PT, DEG_PT)])
    pltpu.sync_copy(row3.at[s, pl.ds(c * BLK, BLK)], row_v)
    pltpu.sync_copy(vals3.at[s, pl.ds(c * BLK, BLK)], vals_v)
    plsc.subcore_barrier()

    def deg_fire(i, _):
        pltpu.async_copy(vals_v.at[i], dacc.at[row_v.at[i]], sg0, add=True)
        return 0

    lax.fori_loop(0, BLK, deg_fire, 0)

    def deg_drain(i, _):
        pltpu.make_async_copy(vals_v.at[i], dacc.at[row_v.at[i]], sg0).wait()
        return 0

    lax.fori_loop(0, BLK, deg_drain, 0)
    plsc.subcore_barrier()
    pltpu.sync_copy(dacc.at[pl.ds(s * DEG_PT, DEG_PT)],
                    deg_out.at[pl.ds(c * DN + s * DEG_PT, DEG_PT)])

    def src_base0(k):
        return k * NP

    def run_core(x_half, p_out):
        for i in range(ROWS_PT // CHUNK):
            pltpu.sync_copy(x_half.at[pl.ds(s * ROWS_PT + i * CHUNK, CHUNK)], g0)
            pltpu.sync_copy(g0, p_out.at[pl.ds(s * ROWS_PT + i * CHUNK, CHUNK)])
        plsc.subcore_barrier()

        def scale(buf, vxbuf):
            def grp(g, _):
                base = g * 16
                for l in range(16):
                    vv = vxbuf[base + l, :]
                    for j in range(FH // 16):
                        buf[base + l, pl.ds(j * 16, 16)] = (
                            buf[base + l, pl.ds(j * 16, 16)] * vv)
                return 0

            lax.fori_loop(0, CHUNK // 16, grp, 0)

        sgs = (sg0, sg1, sg2, sg3)
        for i in range(ROWS_PT // CHUNK):
            pltpu.sync_copy(zbuf, acc.at[pl.ds(s * ROWS_PT + i * CHUNK, CHUNK)])
        plsc.subcore_barrier()

        def round_body(k, _):
            p_src = p_out.at[pl.ds(src_base0(k), NP)]


            for b in range(NCHT // BLK):
                pltpu.sync_copy(col3.at[s, pl.ds(b * BLK, BLK)], col_v)
                pltpu.sync_copy(row3.at[s, pl.ds(b * BLK, BLK)], row_v)
                pltpu.sync_copy(vals3.at[s, pl.ds(b * BLK, BLK)], vals_v)


                bufs = (g0, g1, g2, g3)
                vxb = (vx0, vx1, vx2, vx3b)
                sgs = (sg0, sg1, sg2, sg3)
                sss = (ss0, ss1, ss2, ss3)
                bb = b
                for q in range(4):
                    pltpu.async_copy(p_src.at[col_v.at[q]], bufs[q], sgs[q])
                    pltpu.async_copy(vx3.at[s, bb * BLK + q], vxb[q], sgs[q])

                def pipe(i, _):
                    for q in range(4):
                        lc = 4 * i + q
                        pltpu.make_async_copy(
                            p_src.at[col_v.at[lc]], bufs[q], sgs[q]).wait()
                        pltpu.make_async_copy(
                            vx3.at[s, bb * BLK + lc], vxb[q], sgs[q]).wait()
                        scale(bufs[q], vxb[q])
                        pltpu.async_copy(
                            bufs[q], acc.at[row_v.at[lc]], sss[q], add=True)
                    for q in range(4):
                        lc = 4 * i + q
                        pltpu.make_async_copy(
                            bufs[q], acc.at[row_v.at[lc]], sss[q]).wait()

                        @pl.when(lc + 4 < BLK)
                        def _():
                            pltpu.async_copy(
                                p_src.at[col_v.at[lc + 4]], bufs[q], sgs[q])
                            pltpu.async_copy(
                                vx3.at[s, bb * BLK + lc + 4], vxb[q], sgs[q])

                    return 0

                lax.fori_loop(0, BLK // 4, pipe, 0)

            plsc.subcore_barrier()
            dst = (k + 1) * NP + s * ROWS_PT
            pltpu.sync_copy(acc.at[pl.ds(s * ROWS_PT, ROWS_PT)],
                            p_out.at[pl.ds(dst, ROWS_PT)])
            for i in range(ROWS_PT // CHUNK):
                pltpu.async_copy(
                    zbuf, acc.at[pl.ds(s * ROWS_PT + i * CHUNK, CHUNK)],
                    sgs[i % 4])
            for i in range(ROWS_PT // CHUNK):
                pltpu.make_async_copy(
                    zbuf, acc.at[pl.ds(s * ROWS_PT + i * CHUNK, CHUNK)],
                    sgs[i % 4]).wait()
            plsc.subcore_barrier()
            return 0

        lax.fori_loop(0, K, round_body, 0)

    @pl.when(c == 0)
    def _():
        run_core(xl, pl_out)

    @pl.when(c == 1)
    def _():
        run_core(xr, pr_out)


_HB = 1024


def _softplus(z):
    return jnp.maximum(z, 0.0) + jnp.log1p(jnp.exp(-jnp.abs(z)))


def _hyper_body(x_ref, deg_ref, wp_ref, bp_ref, w1a_ref, w1b_ref, w1c_ref,
                b1_ref, w2_ref, b2_ref, sc_ref, coef_ref, alpha_ref, t_ref):
    pid = pl.program_id(0)
    x = x_ref[...]
    feat = jnp.dot(x, wp_ref[...], preferred_element_type=jnp.float32)
    feat = feat + bp_ref[...]

    d = deg_ref[0, :] + deg_ref[1, :]
    log_deg = jnp.log1p(d)[:, None]

    idx = pid * _HB + lax.broadcasted_iota(jnp.int32, (_HB, 1), 0)
    ntype = (idx >= DRUG_NUM).astype(jnp.float32)

    h = jnp.dot(feat, w1a_ref[...], preferred_element_type=jnp.float32)
    h = h + log_deg * w1b_ref[...] + ntype * w1c_ref[...] + b1_ref[...]
    h = jnp.maximum(h, 0.0)
    raw = jnp.dot(h, w2_ref[...], preferred_element_type=jnp.float32)
    raw = raw + b2_ref[...]

    sc = sc_ref[...]
    a_bias = jnp.where(idx < DRUG_NUM, sc[0, 0], sc[0, 1])
    alpha = _softplus(raw[:, 0:1] + a_bias) + 0.05
    t = _softplus(raw[:, 1:2] + sc[0, 2]) + 0.01
    alpha_ref[...] = alpha
    t_ref[...] = t

    a = jnp.clip(alpha, 0.05, 3.0)
    tt = jnp.clip(t, 0.01, 10.0)
    s = tt / (1.0 + tt)
    base = jnp.exp(-a * jnp.log1p(tt))
    cols = [base]
    rising = jnp.ones_like(a)
    s_pow = jnp.ones_like(a)
    for k in range(1, K + 1):
        rising = rising * (a + (k - 1.0)) / k
        s_pow = s_pow * s
        cols.append(base * rising * s_pow)
    coef_ref[...] = jnp.concatenate(cols, axis=1)


def _hyper_call(x, deg2, wp, bp, w1a, w1b, w1c, b1, w2, b2, scalars):
    full = lambda s: pl.BlockSpec(s, lambda i: tuple(0 for _ in s))
    return pl.pallas_call(
        _hyper_body,
        grid=(pl.cdiv(N, _HB),),
        in_specs=[
            pl.BlockSpec((_HB, F), lambda i: (i, 0)),
            pl.BlockSpec((NC, _HB), lambda i: (0, i)),
            full((F, F)), full((1, F)),
            full((F, F)), full((1, F)), full((1, F)), full((1, F)),
            full((F, F)), full((1, F)),
            full((1, 8)),
        ],
        out_specs=[
            pl.BlockSpec((_HB, K + 1), lambda i: (i, 0)),
            pl.BlockSpec((_HB, 1), lambda i: (i, 0)),
            pl.BlockSpec((_HB, 1), lambda i: (i, 0)),
        ],
        out_shape=[
            jax.ShapeDtypeStruct((N, K + 1), jnp.float32),
            jax.ShapeDtypeStruct((N, 1), jnp.float32),
            jax.ShapeDtypeStruct((N, 1), jnp.float32),
        ],
    )(x, deg2, wp, bp, w1a, w1b, w1c, b1, w2, b2, scalars)


def _wsum_body(coef_ref, pl_ref, pr_ref, out_ref):
    coef = coef_ref[...]
    acc_l = coef[:, 0:1] * pl_ref[0]
    acc_r = coef[:, 0:1] * pr_ref[0]
    for k in range(1, K + 1):
        acc_l = acc_l + coef[:, k:k + 1] * pl_ref[k]
        acc_r = acc_r + coef[:, k:k + 1] * pr_ref[k]
    out_ref[:, 0:FH] = acc_l
    out_ref[:, FH:F] = acc_r


def _wsum_call(coef, p_left, p_right):
    return pl.pallas_call(
        _wsum_body,
        grid=(pl.cdiv(N, _HB),),
        in_specs=[
            pl.BlockSpec((_HB, K + 1), lambda i: (i, 0)),
            pl.BlockSpec((K + 1, _HB, FH), lambda i: (0, i, 0)),
            pl.BlockSpec((K + 1, _HB, FH), lambda i: (0, i, 0)),
        ],
        out_specs=pl.BlockSpec((_HB, F), lambda i: (i, 0)),
        out_shape=jax.ShapeDtypeStruct((N, F), jnp.float32),
    )(coef, p_left, p_right)


def kernel(x, edge_index, edge_vals, W_proj, b_proj, W1, b1, W2, b2,
           alpha_bias_drug, alpha_bias_prot, t_bias):
    row = edge_index[0].astype(jnp.int32)
    col = edge_index[1].astype(jnp.int32)
    vals = edge_vals.astype(jnp.float32)

    pad = E_PAD - E
    zi = jnp.zeros((pad,), jnp.int32)
    row_p = jnp.concatenate([row, zi]).reshape(NS, NCHT, CHUNK)
    col_p = jnp.concatenate([col, zi]).reshape(NS, NCHT, CHUNK)
    vals_p = jnp.concatenate([vals, jnp.zeros((pad,), jnp.float32)]
                             ).reshape(NS, NCHT, CHUNK)

    x_pad = jnp.zeros((NP, F), jnp.float32).at[:N].set(x)
    xl = x_pad[:, :FH]
    xr = x_pad[:, FH:]

    vx = jnp.broadcast_to(vals_p[..., None], (NS, NCHT, CHUNK, 16))
    vx = vx + jnp.zeros((16,), jnp.float32)
    p_left, p_right, deg = _diffuse_kernel(xl, xr, col_p, row_p, vals_p, vx)
    deg2 = deg.reshape(NC, DN)

    wp = jnp.zeros((F, F), jnp.float32).at[:, :W_proj.shape[0]].set(W_proj.T)
    bp = jnp.zeros((1, F), jnp.float32).at[0, :b_proj.shape[0]].set(b_proj)
    nh = W1.shape[0]
    w1a = jnp.zeros((F, F), jnp.float32).at[:W1.shape[1] - 2, :nh].set(W1[:, :-2].T)
    w1b = jnp.zeros((1, F), jnp.float32).at[0, :nh].set(W1[:, -2])
    w1c = jnp.zeros((1, F), jnp.float32).at[0, :nh].set(W1[:, -1])
    b1p = jnp.zeros((1, F), jnp.float32).at[0, :nh].set(b1)
    w2 = jnp.zeros((F, F), jnp.float32).at[:nh, :2].set(W2.T)
    b2p = jnp.zeros((1, F), jnp.float32).at[0, :2].set(b2)
    scalars = jnp.zeros((1, 8), jnp.float32)
    scalars = scalars.at[0, 0].set(alpha_bias_drug)
    scalars = scalars.at[0, 1].set(alpha_bias_prot)
    scalars = scalars.at[0, 2].set(t_bias)

    coef, alpha2, t2 = _hyper_call(x, deg2, wp, bp, w1a, w1b, w1c, b1p,
                                   w2, b2p, scalars)

    out = _wsum_call(coef,
                     p_left.reshape(K + 1, NP, FH),
                     p_right.reshape(K + 1, NP, FH))
    return (out, alpha2[:, 0], t2[:, 0])
